# baseline (speedup 1.0000x reference)
import functools

import jax
import jax.numpy as jnp
from jax import lax
from jax.experimental import pallas as pl
from jax.experimental.pallas import tpu as pltpu
from jax.experimental.pallas import tpu_sc as plsc

NFREQ = 8193
SEG = 512
NMAIN = 16 * SEG
BUF = 8208
UNROLL = 4
NROWS = 512
NWORKERS = 32
ROWS_PER_W = NROWS // NWORKERS
SCALE = jnp.float32(0.5)


def _rsqrt_vec(t):
    i = plsc.bitcast(t, jnp.int32)
    r = plsc.bitcast(jnp.int32(0x5F3759DF) - (i >> 1), jnp.float32)
    for _ in range(3):
        r = r * (jnp.float32(1.5) - jnp.float32(0.5) * t * r * r)
    return r


_MESH = plsc.VectorSubcoreMesh(core_axis_name="c", subcore_axis_name="s")


@functools.partial(
    pl.kernel,
    mesh=_MESH,
    compiler_params=pltpu.CompilerParams(
        needs_layout_passes=False, use_tc_tiling_on_sc=False
    ),
    out_type=(
        jax.ShapeDtypeStruct((NROWS,), jnp.float32),
        jax.ShapeDtypeStruct((NROWS,), jnp.float32),
    ),
    scratch_types=[
        pltpu.VMEM((BUF,), jnp.float32),
        pltpu.VMEM((BUF,), jnp.float32),
        pltpu.VMEM((BUF,), jnp.float32),
        pltpu.VMEM((BUF,), jnp.float32),
        pltpu.VMEM((NMAIN,), jnp.float32),
        pltpu.VMEM((NMAIN,), jnp.float32),
        pltpu.VMEM((16,), jnp.float32),
        pltpu.VMEM((16,), jnp.float32),
        pltpu.VMEM((16,), jnp.int32),
        pltpu.VMEM((16,), jnp.float32),
        pltpu.VMEM((16,), jnp.float32),
        pltpu.SemaphoreType.DMA,
        pltpu.SemaphoreType.DMA,
        pltpu.SemaphoreType.DMA,
        pltpu.SemaphoreType.DMA,
    ],
)
def _chisq_sc(h_hbm, s_hbm, snr_hbm, chisq_hbm, h_a, h_b, s_a, s_b, xbuf,
              ybuf, soxbuf, soybuf, ebuf, snrbuf, chibuf,
              sem_ha, sem_sa, sem_hb, sem_sb):
    wid = lax.axis_index("s") * 2 + lax.axis_index("c")
    row0 = wid * ROWS_PER_W
    lanes = lax.iota(jnp.int32, 16)
    fzero = jnp.zeros((16,), jnp.float32)
    seg_base = lanes << 9

    def start_row(row, hbuf, sbuf, sem_h, sem_s):
        pltpu.make_async_copy(h_hbm.at[row], hbuf.at[pl.ds(0, NFREQ)], sem_h).start()
        pltpu.make_async_copy(s_hbm.at[row], sbuf.at[pl.ds(0, NFREQ)], sem_s).start()

    def wait_row(row, hbuf, sbuf, sem_h, sem_s):
        pltpu.make_async_copy(h_hbm.at[row], hbuf.at[pl.ds(0, NFREQ)], sem_h).wait()
        pltpu.make_async_copy(s_hbm.at[row], sbuf.at[pl.ds(0, NFREQ)], sem_s).wait()

    def process_row(hbuf, sbuf):
        def grp(gi, c):
            acc_x, acc_y = c
            i0 = gi * UNROLL
            for u in range(UNROLL):
                i = i0 + u
                idx = seg_base + i
                hv = plsc.load_gather(hbuf, [idx])
                sv = plsc.load_gather(sbuf, [idx])
                acc_x = acc_x + hv * hv * SCALE
                acc_y = acc_y + jnp.abs(hv * sv) * SCALE
                xbuf[pl.ds(i * 16, 16)] = acc_x
                ybuf[pl.ds(i * 16, 16)] = acc_y
            return acc_x, acc_y

        acc_x, acc_y = lax.fori_loop(0, SEG // UNROLL, grp, (fzero, fzero))

        cs_x = plsc.cumsum(acc_x)
        cs_y = plsc.cumsum(acc_y)
        soxbuf[...] = cs_x - acc_x
        soybuf[...] = cs_y - acc_y
        hl = plsc.load_gather(hbuf, [jnp.full((16,), NFREQ - 1, jnp.int32)])
        sl = plsc.load_gather(sbuf, [jnp.full((16,), NFREQ - 1, jnp.int32)])
        total = jnp.max(cs_x) + hl * hl * SCALE
        sum_y = jnp.max(cs_y) + jnp.abs(hl * sl) * SCALE

        t_vec = lanes.astype(jnp.float32) * (total * jnp.float32(1.0 / 16.0))

        def translate(f):
            return ((f & 511) << 4) | (f >> 9)

        def bs_body(_, c):
            lo, hi = c
            mid = (lo + hi) >> 1
            vals = (plsc.load_gather(xbuf, [translate(mid)])
                    + plsc.load_gather(soxbuf, [mid >> 9]))
            pred = vals <= t_vec
            return jnp.where(pred, mid + 1, lo), jnp.where(pred, hi, mid)

        e, _ = lax.fori_loop(
            0, 13, bs_body,
            (jnp.zeros((16,), jnp.int32), jnp.full((16,), NMAIN, jnp.int32)),
        )

        def y_at(f):
            fc = jnp.minimum(f, NMAIN - 1)
            v = (plsc.load_gather(ybuf, [translate(fc)])
                 + plsc.load_gather(soybuf, [fc >> 9]))
            return jnp.where(f >= NMAIN, sum_y, v)

        left = y_at(e)
        ebuf[...] = e
        ridx = plsc.load_gather(ebuf, [jnp.minimum(lanes + 1, 15)])
        right = y_at(jnp.where(lanes == 15, NMAIN, ridx))

        rs = _rsqrt_vec(total)
        spb = (right - left) * rs
        snr_splat = sum_y * rs
        d = spb - snr_splat * jnp.float32(1.0 / 16.0)
        chi = jnp.sum(d * d) * jnp.float32(16.0 / 15.0)
        return snr_splat, chi

    start_row(row0, h_a, s_a, sem_ha, sem_sa)

    def row_pair(g, carry):
        snr_res, chi_res = carry
        row_a = row0 + 2 * g
        row_b = row_a + 1
        start_row(row_b, h_b, s_b, sem_hb, sem_sb)
        wait_row(row_a, h_a, s_a, sem_ha, sem_sa)
        snr_v, chi_v = process_row(h_a, s_a)
        snr_res = jnp.where(lanes == 2 * g, snr_v, snr_res)
        chi_res = jnp.where(lanes == 2 * g, chi_v, chi_res)
        row_n = jnp.minimum(row_a + 2, row0 + ROWS_PER_W - 1)
        start_row(row_n, h_a, s_a, sem_ha, sem_sa)
        wait_row(row_b, h_b, s_b, sem_hb, sem_sb)
        snr_v, chi_v = process_row(h_b, s_b)
        snr_res = jnp.where(lanes == 2 * g + 1, snr_v, snr_res)
        chi_res = jnp.where(lanes == 2 * g + 1, chi_v, chi_res)
        return snr_res, chi_res

    snr_res, chi_res = lax.fori_loop(
        0, ROWS_PER_W // 2, row_pair, (fzero, fzero)
    )
    wait_row(row0 + ROWS_PER_W - 1, h_a, s_a, sem_ha, sem_sa)

    snrbuf[...] = snr_res
    chibuf[...] = chi_res
    pltpu.sync_copy(snrbuf, snr_hbm.at[pl.ds(row0, ROWS_PER_W)])
    pltpu.sync_copy(chibuf, chisq_hbm.at[pl.ds(row0, ROWS_PER_W)])


def kernel(htilde, stilde):
    b, c, f = htilde.shape
    snr, chisq = _chisq_sc(htilde.reshape(b * c, f), stilde.reshape(b * c, f))
    return snr.reshape(b, c), chisq.reshape(b, c)


# speedup vs baseline: 1.8165x; 1.8165x over previous
import functools

import jax
import jax.numpy as jnp
from jax import lax
from jax.experimental import pallas as pl
from jax.experimental.pallas import tpu as pltpu
from jax.experimental.pallas import tpu_sc as plsc

NFREQ = 8193
SEG = 513
PAD = 16 * SEG
BUF = PAD
UNROLL = 8
NROWS = 512
NWORKERS = 32
ROWS_PER_W = NROWS // NWORKERS
SCALE = jnp.float32(0.5)
MAGIC = 8177


def _rsqrt_vec(t):
    i = plsc.bitcast(t, jnp.int32)
    r = plsc.bitcast(jnp.int32(0x5F3759DF) - (i >> 1), jnp.float32)
    for _ in range(3):
        r = r * (jnp.float32(1.5) - jnp.float32(0.5) * t * r * r)
    return r


_MESH = plsc.VectorSubcoreMesh(core_axis_name="c", subcore_axis_name="s")


@functools.partial(
    pl.kernel,
    mesh=_MESH,
    compiler_params=pltpu.CompilerParams(
        needs_layout_passes=False, use_tc_tiling_on_sc=False
    ),
    out_type=(
        jax.ShapeDtypeStruct((NROWS,), jnp.float32),
        jax.ShapeDtypeStruct((NROWS,), jnp.float32),
    ),
    scratch_types=[
        pltpu.VMEM((BUF,), jnp.float32),
        pltpu.VMEM((BUF,), jnp.float32),
        pltpu.VMEM((BUF,), jnp.float32),
        pltpu.VMEM((BUF,), jnp.float32),
        pltpu.VMEM((PAD,), jnp.float32),
        pltpu.VMEM((PAD,), jnp.float32),
        pltpu.VMEM((16,), jnp.float32),
        pltpu.VMEM((16,), jnp.float32),
        pltpu.VMEM((16,), jnp.int32),
        pltpu.VMEM((16,), jnp.float32),
        pltpu.VMEM((16,), jnp.float32),
        pltpu.SemaphoreType.DMA,
        pltpu.SemaphoreType.DMA,
        pltpu.SemaphoreType.DMA,
        pltpu.SemaphoreType.DMA,
    ],
)
def _chisq_sc(h_hbm, s_hbm, snr_hbm, chisq_hbm, h_a, h_b, s_a, s_b, xbuf,
              ybuf, soxbuf, soybuf, ebuf, snrbuf, chibuf,
              sem_ha, sem_sa, sem_hb, sem_sb):
    wid = lax.axis_index("s") * 2 + lax.axis_index("c")
    row0 = wid * ROWS_PER_W
    lanes = lax.iota(jnp.int32, 16)
    fzero = jnp.zeros((16,), jnp.float32)
    seg_base = lanes * SEG

    for buf in (h_a, h_b, s_a, s_b):
        buf[pl.ds(PAD - 16, 16)] = fzero

    def start_row(row, hbuf, sbuf, sem_h, sem_s):
        pltpu.make_async_copy(h_hbm.at[row], hbuf.at[pl.ds(0, NFREQ)], sem_h).start()
        pltpu.make_async_copy(s_hbm.at[row], sbuf.at[pl.ds(0, NFREQ)], sem_s).start()

    def wait_row(row, hbuf, sbuf, sem_h, sem_s):
        pltpu.make_async_copy(h_hbm.at[row], hbuf.at[pl.ds(0, NFREQ)], sem_h).wait()
        pltpu.make_async_copy(s_hbm.at[row], sbuf.at[pl.ds(0, NFREQ)], sem_s).wait()

    def process_row(hbuf, sbuf):
        def acc_step(i, c):
            acc_x, acc_y = c
            idx = seg_base + i
            hv = plsc.load_gather(hbuf, [idx])
            sv = plsc.load_gather(sbuf, [idx])
            acc_x = acc_x + hv * hv
            acc_y = acc_y + jnp.abs(hv * sv)
            xbuf[pl.ds(i * 16, 16)] = acc_x
            ybuf[pl.ds(i * 16, 16)] = acc_y
            return acc_x, acc_y

        acc = plsc.parallel_loop(0, SEG - 1, 1, unroll=UNROLL, carry=(fzero, fzero))(
            acc_step
        )
        acc_x, acc_y = acc_step(jnp.int32(SEG - 1), acc)

        cs_x = plsc.cumsum(acc_x)
        cs_y = plsc.cumsum(acc_y)
        soxbuf[...] = cs_x - acc_x
        soybuf[...] = cs_y - acc_y
        total = jnp.max(cs_x) + fzero
        sum_y = jnp.max(cs_y) + fzero

        def translate(f):
            q = (f * MAGIC) >> 22
            i = f - q * SEG
            return (i << 4) + q, q

        t_vec = lanes.astype(jnp.float32) * (total * jnp.float32(1.0 / 16.0))

        def bs_body(_, c):
            lo, hi = c
            mid = (lo + hi) >> 1
            pos, q = translate(mid)
            vals = (plsc.load_gather(xbuf, [pos])
                    + plsc.load_gather(soxbuf, [q]))
            pred = vals <= t_vec
            return jnp.where(pred, mid + 1, lo), jnp.where(pred, hi, mid)

        lo, _ = lax.fori_loop(
            0, 14, bs_body,
            (jnp.zeros((16,), jnp.int32), jnp.full((16,), PAD, jnp.int32)),
        )
        e = jnp.minimum(lo, NFREQ - 1)

        def y_at(f):
            pos, q = translate(f)
            return (plsc.load_gather(ybuf, [pos])
                    + plsc.load_gather(soybuf, [q]))

        left = y_at(e)
        ebuf[...] = e
        ridx = plsc.load_gather(ebuf, [jnp.minimum(lanes + 1, 15)])
        right = y_at(jnp.where(lanes == 15, NFREQ - 1, ridx))

        rs = _rsqrt_vec(total * SCALE) * SCALE
        spb = (right - left) * rs
        snr_splat = sum_y * rs
        d = spb - snr_splat * jnp.float32(1.0 / 16.0)
        chi = jnp.sum(d * d) * jnp.float32(16.0 / 15.0)
        return snr_splat, chi

    start_row(row0, h_a, s_a, sem_ha, sem_sa)

    def row_pair(g, carry):
        snr_res, chi_res = carry
        row_a = row0 + 2 * g
        row_b = row_a + 1
        start_row(row_b, h_b, s_b, sem_hb, sem_sb)
        wait_row(row_a, h_a, s_a, sem_ha, sem_sa)
        snr_v, chi_v = process_row(h_a, s_a)
        snr_res = jnp.where(lanes == 2 * g, snr_v, snr_res)
        chi_res = jnp.where(lanes == 2 * g, chi_v, chi_res)
        row_n = jnp.minimum(row_a + 2, row0 + ROWS_PER_W - 1)
        start_row(row_n, h_a, s_a, sem_ha, sem_sa)
        wait_row(row_b, h_b, s_b, sem_hb, sem_sb)
        snr_v, chi_v = process_row(h_b, s_b)
        snr_res = jnp.where(lanes == 2 * g + 1, snr_v, snr_res)
        chi_res = jnp.where(lanes == 2 * g + 1, chi_v, chi_res)
        return snr_res, chi_res

    snr_res, chi_res = lax.fori_loop(
        0, ROWS_PER_W // 2, row_pair, (fzero, fzero)
    )
    wait_row(row0 + ROWS_PER_W - 1, h_a, s_a, sem_ha, sem_sa)

    snrbuf[...] = snr_res
    chibuf[...] = chi_res
    pltpu.sync_copy(snrbuf, snr_hbm.at[pl.ds(row0, ROWS_PER_W)])
    pltpu.sync_copy(chibuf, chisq_hbm.at[pl.ds(row0, ROWS_PER_W)])


def kernel(htilde, stilde):
    b, c, f = htilde.shape
    snr, chisq = _chisq_sc(htilde.reshape(b * c, f), stilde.reshape(b * c, f))
    return snr.reshape(b, c), chisq.reshape(b, c)


# speedup vs baseline: 2.8131x; 1.5486x over previous
import functools

import jax
import jax.numpy as jnp
from jax import lax
from jax.experimental import pallas as pl
from jax.experimental.pallas import tpu as pltpu
from jax.experimental.pallas import tpu_sc as plsc

NFREQ = 8193
SEG = 513
PAD = 16 * SEG
BUF = PAD
UNROLL = 8
NROWS = 512
NWORKERS = 32
ROWS_PER_W = NROWS // NWORKERS
SCALE = jnp.float32(0.5)
MAGIC = 8177


def _rsqrt_vec(t):
    i = plsc.bitcast(t, jnp.int32)
    r = plsc.bitcast(jnp.int32(0x5F3759DF) - (i >> 1), jnp.float32)
    for _ in range(3):
        r = r * (jnp.float32(1.5) - jnp.float32(0.5) * t * r * r)
    return r


_MESH = plsc.VectorSubcoreMesh(core_axis_name="c", subcore_axis_name="s")


@functools.partial(
    pl.kernel,
    mesh=_MESH,
    compiler_params=pltpu.CompilerParams(
        needs_layout_passes=False, use_tc_tiling_on_sc=False
    ),
    out_type=(
        jax.ShapeDtypeStruct((NROWS,), jnp.float32),
        jax.ShapeDtypeStruct((NROWS,), jnp.float32),
    ),
    scratch_types=[
        pltpu.VMEM((BUF,), jnp.float32),
        pltpu.VMEM((BUF,), jnp.float32),
        pltpu.VMEM((BUF,), jnp.float32),
        pltpu.VMEM((BUF,), jnp.float32),
        pltpu.VMEM((PAD,), jnp.float32),
        pltpu.VMEM((PAD,), jnp.float32),
        pltpu.VMEM((16,), jnp.float32),
        pltpu.VMEM((16,), jnp.float32),
        pltpu.VMEM((16,), jnp.int32),
        pltpu.VMEM((16,), jnp.float32),
        pltpu.VMEM((16,), jnp.float32),
        pltpu.SemaphoreType.DMA,
        pltpu.SemaphoreType.DMA,
        pltpu.SemaphoreType.DMA,
        pltpu.SemaphoreType.DMA,
    ],
)
def _chisq_sc(h_hbm, s_hbm, snr_hbm, chisq_hbm, h_a, h_b, s_a, s_b, xbuf,
              ybuf, soxbuf, soybuf, ebuf, snrbuf, chibuf,
              sem_ha, sem_sa, sem_hb, sem_sb):
    wid = lax.axis_index("s") * 2 + lax.axis_index("c")
    row0 = wid * ROWS_PER_W
    lanes = lax.iota(jnp.int32, 16)
    fzero = jnp.zeros((16,), jnp.float32)
    seg_base = lanes * SEG

    for buf in (h_a, h_b, s_a, s_b):
        buf[pl.ds(PAD - 16, 16)] = fzero

    def start_row(row, hbuf, sbuf, sem_h, sem_s):
        b, c = row >> 1, row & 1
        pltpu.make_async_copy(h_hbm.at[b, c], hbuf.at[pl.ds(0, NFREQ)], sem_h).start()
        pltpu.make_async_copy(s_hbm.at[b, c], sbuf.at[pl.ds(0, NFREQ)], sem_s).start()

    def wait_row(row, hbuf, sbuf, sem_h, sem_s):
        b, c = row >> 1, row & 1
        pltpu.make_async_copy(h_hbm.at[b, c], hbuf.at[pl.ds(0, NFREQ)], sem_h).wait()
        pltpu.make_async_copy(s_hbm.at[b, c], sbuf.at[pl.ds(0, NFREQ)], sem_s).wait()

    def process_row(hbuf, sbuf):
        def acc_step(i, c):
            acc_x, acc_y = c
            idx = seg_base + i
            hv = plsc.load_gather(hbuf, [idx])
            sv = plsc.load_gather(sbuf, [idx])
            acc_x = acc_x + hv * hv
            acc_y = acc_y + jnp.abs(hv * sv)
            xbuf[pl.ds(i * 16, 16)] = acc_x
            ybuf[pl.ds(i * 16, 16)] = acc_y
            return acc_x, acc_y

        acc = plsc.parallel_loop(0, SEG - 1, 1, unroll=UNROLL, carry=(fzero, fzero))(
            acc_step
        )
        acc_x, acc_y = acc_step(jnp.int32(SEG - 1), acc)

        cs_x = plsc.cumsum(acc_x)
        cs_y = plsc.cumsum(acc_y)
        soxbuf[...] = cs_x - acc_x
        soybuf[...] = cs_y - acc_y
        total = jnp.max(cs_x) + fzero
        sum_y = jnp.max(cs_y) + fzero

        def translate(f):
            q = (f * MAGIC) >> 22
            i = f - q * SEG
            return (i << 4) + q, q

        t_vec = lanes.astype(jnp.float32) * (total * jnp.float32(1.0 / 16.0))

        def bs_body(_, c):
            lo, hi = c
            mid = (lo + hi) >> 1
            pos, q = translate(mid)
            vals = (plsc.load_gather(xbuf, [pos])
                    + plsc.load_gather(soxbuf, [q]))
            pred = vals <= t_vec
            return jnp.where(pred, mid + 1, lo), jnp.where(pred, hi, mid)

        lo, _ = lax.fori_loop(
            0, 14, bs_body,
            (jnp.zeros((16,), jnp.int32), jnp.full((16,), PAD, jnp.int32)),
        )
        e = jnp.minimum(lo, NFREQ - 1)

        def y_at(f):
            pos, q = translate(f)
            return (plsc.load_gather(ybuf, [pos])
                    + plsc.load_gather(soybuf, [q]))

        left = y_at(e)
        ebuf[...] = e
        ridx = plsc.load_gather(ebuf, [jnp.minimum(lanes + 1, 15)])
        right = y_at(jnp.where(lanes == 15, NFREQ - 1, ridx))

        rs = _rsqrt_vec(total * SCALE) * SCALE
        spb = (right - left) * rs
        snr_splat = sum_y * rs
        d = spb - snr_splat * jnp.float32(1.0 / 16.0)
        chi = jnp.sum(d * d) * jnp.float32(16.0 / 15.0)
        return snr_splat, chi

    start_row(row0, h_a, s_a, sem_ha, sem_sa)

    def row_pair(g, carry):
        snr_res, chi_res = carry
        row_a = row0 + 2 * g
        row_b = row_a + 1
        start_row(row_b, h_b, s_b, sem_hb, sem_sb)
        wait_row(row_a, h_a, s_a, sem_ha, sem_sa)
        snr_v, chi_v = process_row(h_a, s_a)
        snr_res = jnp.where(lanes == 2 * g, snr_v, snr_res)
        chi_res = jnp.where(lanes == 2 * g, chi_v, chi_res)
        row_n = jnp.minimum(row_a + 2, row0 + ROWS_PER_W - 1)
        start_row(row_n, h_a, s_a, sem_ha, sem_sa)
        wait_row(row_b, h_b, s_b, sem_hb, sem_sb)
        snr_v, chi_v = process_row(h_b, s_b)
        snr_res = jnp.where(lanes == 2 * g + 1, snr_v, snr_res)
        chi_res = jnp.where(lanes == 2 * g + 1, chi_v, chi_res)
        return snr_res, chi_res

    snr_res, chi_res = lax.fori_loop(
        0, ROWS_PER_W // 2, row_pair, (fzero, fzero)
    )
    wait_row(row0 + ROWS_PER_W - 1, h_a, s_a, sem_ha, sem_sa)

    snrbuf[...] = snr_res
    chibuf[...] = chi_res
    pltpu.sync_copy(snrbuf, snr_hbm.at[pl.ds(row0, ROWS_PER_W)])
    pltpu.sync_copy(chibuf, chisq_hbm.at[pl.ds(row0, ROWS_PER_W)])


def kernel(htilde, stilde):
    b, c, _ = htilde.shape
    snr, chisq = _chisq_sc(htilde, stilde)
    return snr.reshape(b, c), chisq.reshape(b, c)
